# baseline (speedup 1.0000x reference)
import jax
import jax.numpy as jnp
from jax import lax
from jax.experimental import pallas as pl
from jax.experimental.pallas import tpu as pltpu
from jax.experimental.pallas import tpu_sc as plsc

BATCH = 10000
D = 128
DEG = 16
NREL = 3
NC, NS, L = 2, 16, 16
NW = NC * NS
BPW = 320
NPAD = NW * BPW
CH = 8
NCH = BPW // CH
NVR = D // L
SELF_CH = 64
NSELF = BPW // SELF_CH


def _sc_body(alpha_hbm, nodes_hbm, neigh_hbm, feat_hbm, self_out, wagg_out,
             alpha_v, w_v, nodes_v, neigh_v, self_v, wagg_v, st_v, sem, sem2):
    wid = lax.axis_index("s") * NC + lax.axis_index("c")
    base = wid * BPW

    pltpu.sync_copy(nodes_hbm.at[wid], nodes_v)
    self_dmas = []
    for s in range(NSELF):
        self_dmas.append(pltpu.async_copy(
            feat_hbm.at[nodes_v.at[pl.ds(s * SELF_CH, SELF_CH)]],
            self_v.at[pl.ds(s * SELF_CH, SELF_CH)], sem2))

    pltpu.sync_copy(alpha_hbm, alpha_v)
    for i in range(NVR):
        sl = pl.ds(D + i * L, L)
        a0, a1, a2 = alpha_v[0, sl], alpha_v[1, sl], alpha_v[2, sl]
        m = jnp.maximum(jnp.maximum(a0, a1), a2)
        e0, e1, e2 = jnp.exp(a0 - m), jnp.exp(a1 - m), jnp.exp(a2 - m)
        inv = (1.0 / DEG) / (e0 + e1 + e2)
        osl = pl.ds(i * L, L)
        w_v[0, osl] = e0 * inv
        w_v[1, osl] = e1 * inv
        w_v[2, osl] = e2 * inv

    zero = jnp.zeros((L,), jnp.float32)

    def zbody(i, c):
        for k in range(NVR):
            wagg_v[i, pl.ds(k * L, L)] = zero
        return c
    lax.fori_loop(0, BPW, zbody, 0)

    def rel_body(r, carry):
        pltpu.sync_copy(neigh_hbm.at[r, wid], neigh_v)
        wk = tuple(w_v[r, pl.ds(k * L, L)] for k in range(NVR))

        def chunk_body(c, cc):
            off = pl.multiple_of(c * (CH * DEG), CH * DEG)
            pltpu.async_copy(
                feat_hbm.at[neigh_v.at[pl.ds(off, CH * DEG)]], st_v, sem).wait()

            def center_body(j, jc):
                row0 = j * DEG
                ci = c * CH + j
                for k in range(NVR):
                    sl = pl.ds(k * L, L)
                    vals = [st_v[row0 + t, sl] for t in range(DEG)]
                    while len(vals) > 1:
                        vals = [vals[2 * i] + vals[2 * i + 1]
                                for i in range(len(vals) // 2)]
                    wagg_v[ci, sl] = wagg_v[ci, sl] + vals[0] * wk[k]
                return jc
            lax.fori_loop(0, CH, center_body, 0)
            return cc
        lax.fori_loop(0, NCH, chunk_body, 0)
        return carry
    lax.fori_loop(0, NREL, rel_body, 0)

    for cp in self_dmas:
        cp.wait()
    pltpu.sync_copy(self_v, self_out.at[pl.ds(base, BPW)])
    pltpu.sync_copy(wagg_v, wagg_out.at[pl.ds(base, BPW)])


def _pad_idx(x, n_rows):
    x = x.astype(jnp.int32)
    pad = n_rows - x.shape[0]
    cfg = [(0, pad)] + [(0, 0)] * (x.ndim - 1)
    return jnp.pad(x, cfg)


def kernel(features, alpha, nodes, neigh1, neigh2, neigh3):
    alpha_t = alpha.T.astype(jnp.float32)
    nodes_p = _pad_idx(nodes, NPAD).reshape(NW, BPW)
    neigh_p = jnp.stack([
        _pad_idx(n, NPAD).reshape(NW, BPW * DEG)
        for n in (neigh1, neigh2, neigh3)])

    mesh = plsc.VectorSubcoreMesh(core_axis_name="c", subcore_axis_name="s")
    f = pl.kernel(
        _sc_body,
        out_type=(jax.ShapeDtypeStruct((NPAD, D), jnp.float32),
                  jax.ShapeDtypeStruct((NPAD, D), jnp.float32)),
        mesh=mesh,
        scratch_types=(
            pltpu.VMEM((NREL, 2 * D), jnp.float32),
            pltpu.VMEM((NREL, D), jnp.float32),
            pltpu.VMEM((BPW,), jnp.int32),
            pltpu.VMEM((BPW * DEG,), jnp.int32),
            pltpu.VMEM((BPW, D), jnp.float32),
            pltpu.VMEM((BPW, D), jnp.float32),
            pltpu.VMEM((CH * DEG, D), jnp.float32),
            pltpu.SemaphoreType.DMA,
            pltpu.SemaphoreType.DMA,
        ),
    )
    self_o, wagg_o = f(alpha_t, nodes_p, neigh_p, features)
    return jnp.concatenate(
        [self_o[:BATCH], self_o[:BATCH], wagg_o[:BATCH]], axis=1)


# speedup vs baseline: 1.1689x; 1.1689x over previous
import jax
import jax.numpy as jnp
from jax import lax
from jax.experimental import pallas as pl
from jax.experimental.pallas import tpu as pltpu
from jax.experimental.pallas import tpu_sc as plsc

BATCH = 10000
D = 128
DEG = 16
NREL = 3
NC, NS, L = 2, 16, 16
NW = NC * NS
BPW = 320
NPAD = NW * BPW
CH = 8
NCH = BPW // CH
TOT = NREL * NCH
NBUF = 3
NVR = D // L


def _sc_body(alpha_hbm, nodes_hbm, neigh_hbm, feat_hbm, self_out, wagg_out,
             alpha_v, w_v, nodes_v, neigh_v, wagg_v,
             st0, st1, st2, sem0, sem1, sem2):
    sts = (st0, st1, st2)
    sems = (sem0, sem1, sem2)
    wid = lax.axis_index("s") * NC + lax.axis_index("c")
    base = wid * BPW

    pltpu.sync_copy(nodes_hbm.at[wid], nodes_v)
    self_plan = ((0, 128, 0), (128, 128, 1), (256, 64, 2))
    cps = [pltpu.async_copy(
        feat_hbm.at[nodes_v.at[pl.ds(s_off, s_len)]],
        sts[b].at[pl.ds(0, s_len)], sems[b]) for s_off, s_len, b in self_plan]
    for cp, (s_off, s_len, b) in zip(cps, self_plan):
        cp.wait()
        pltpu.sync_copy(sts[b].at[pl.ds(0, s_len)],
                        self_out.at[pl.ds(base + s_off, s_len)])

    pltpu.sync_copy(alpha_hbm, alpha_v)
    for i in range(NVR):
        sl = pl.ds(D + i * L, L)
        a0, a1, a2 = alpha_v[0, sl], alpha_v[1, sl], alpha_v[2, sl]
        m = jnp.maximum(jnp.maximum(a0, a1), a2)
        e0, e1, e2 = jnp.exp(a0 - m), jnp.exp(a1 - m), jnp.exp(a2 - m)
        inv = (1.0 / DEG) / (e0 + e1 + e2)
        osl = pl.ds(i * L, L)
        w_v[0, osl] = e0 * inv
        w_v[1, osl] = e1 * inv
        w_v[2, osl] = e2 * inv

    zero = jnp.zeros((L,), jnp.float32)

    def zbody(i, c):
        for k in range(NVR):
            wagg_v[i, pl.ds(k * L, L)] = zero
        return c
    lax.fori_loop(0, BPW, zbody, 0)

    pltpu.sync_copy(neigh_hbm.at[wid], neigh_v)

    def fire(c, b):
        off = pl.multiple_of(c * (CH * DEG), CH * DEG)
        return pltpu.async_copy(
            feat_hbm.at[neigh_v.at[pl.ds(off, CH * DEG)]], sts[b], sems[b])

    def process(c, st):
        r = c // NCH
        wk = tuple(w_v[r, pl.ds(k * L, L)] for k in range(NVR))
        c0 = (c % NCH) * CH

        def center_body(j, jc):
            row0 = j * DEG
            ci = c0 + j
            for k in range(NVR):
                sl = pl.ds(k * L, L)
                vals = [st[row0 + t, sl] for t in range(DEG)]
                while len(vals) > 1:
                    vals = [vals[2 * i] + vals[2 * i + 1]
                            for i in range(len(vals) // 2)]
                wagg_v[ci, sl] = wagg_v[ci, sl] + vals[0] * wk[k]
            return jc
        lax.fori_loop(0, CH, center_body, 0)

    for b in range(NBUF):
        fire(b, b)

    def main_body(p, carry):
        for b in range(NBUF):
            c = p * NBUF + b
            _wait_chunk(feat_hbm, sts[b], sems[b])
            process(c, sts[b])
            fire(c + NBUF, b)
        return carry
    lax.fori_loop(0, (TOT - NBUF) // NBUF, main_body, 0)

    for b in range(NBUF):
        _wait_chunk(feat_hbm, sts[b], sems[b])
        process(TOT - NBUF + b, sts[b])

    pltpu.sync_copy(wagg_v, wagg_out.at[pl.ds(base, BPW)])


def _wait_chunk(feat_hbm, st, sem):
    pltpu.make_async_copy(feat_hbm.at[pl.ds(0, CH * DEG)], st, sem).wait()


def _pad_idx(x, n_rows):
    x = x.astype(jnp.int32)
    pad = n_rows - x.shape[0]
    cfg = [(0, pad)] + [(0, 0)] * (x.ndim - 1)
    return jnp.pad(x, cfg)


def kernel(features, alpha, nodes, neigh1, neigh2, neigh3):
    alpha_t = alpha.T.astype(jnp.float32)
    nodes_p = _pad_idx(nodes, NPAD).reshape(NW, BPW)
    neigh_p = jnp.stack([
        _pad_idx(n, NPAD).reshape(NW, BPW * DEG)
        for n in (neigh1, neigh2, neigh3)], axis=1)
    neigh_p = neigh_p.reshape(NW, NREL * BPW * DEG)

    mesh = plsc.VectorSubcoreMesh(core_axis_name="c", subcore_axis_name="s")
    f = pl.kernel(
        _sc_body,
        out_type=(jax.ShapeDtypeStruct((NPAD, D), jnp.float32),
                  jax.ShapeDtypeStruct((NPAD, D), jnp.float32)),
        mesh=mesh,
        scratch_types=(
            pltpu.VMEM((NREL, 2 * D), jnp.float32),
            pltpu.VMEM((NREL, D), jnp.float32),
            pltpu.VMEM((BPW,), jnp.int32),
            pltpu.VMEM((NREL * BPW * DEG,), jnp.int32),
            pltpu.VMEM((BPW, D), jnp.float32),
            pltpu.VMEM((CH * DEG, D), jnp.float32),
            pltpu.VMEM((CH * DEG, D), jnp.float32),
            pltpu.VMEM((CH * DEG, D), jnp.float32),
            pltpu.SemaphoreType.DMA,
            pltpu.SemaphoreType.DMA,
            pltpu.SemaphoreType.DMA,
        ),
    )
    self_o, wagg_o = f(alpha_t, nodes_p, neigh_p, features)
    return jnp.concatenate(
        [self_o[:BATCH], self_o[:BATCH], wagg_o[:BATCH]], axis=1)


# speedup vs baseline: 1.1887x; 1.0169x over previous
import jax
import jax.numpy as jnp
from jax import lax
from jax.experimental import pallas as pl
from jax.experimental.pallas import tpu as pltpu
from jax.experimental.pallas import tpu_sc as plsc

BATCH = 10000
D = 128
DEG = 16
NREL = 3
NC, NS, L = 2, 16, 16
NW = NC * NS
BPW = 320
NPAD = NW * BPW
CH = 8
NCH = BPW // CH
TOT = NREL * NCH
NBUF = 2
NVR = D // L
SCH = BPW // 5


def _sc_body(alpha_hbm, nodes_hbm, neigh_hbm, feat_hbm,
             self_out, wagg_out,
             alpha_v, w_v, nodes_v, neigh_v, wagg_v,
             st0, st1, sf0, sf1, sem0, sem1):
    sts = (st0, st1)
    sfs = (sf0, sf1)
    sems = (sem0, sem1)
    wid = lax.axis_index("s") * NC + lax.axis_index("c")
    base = wid * BPW

    pltpu.sync_copy(nodes_hbm.at[wid], nodes_v)
    self_plan = [(g * SCH, g % 2) for g in range(BPW // SCH)]
    for g in range(0, len(self_plan), 2):
        grp = self_plan[g:g + 2]
        cps = [pltpu.async_copy(
            feat_hbm.at[nodes_v.at[pl.ds(s_off, SCH)]], sfs[b], sems[b])
            for s_off, b in grp]
        for cp, (s_off, b) in zip(cps, grp):
            cp.wait()
            pltpu.sync_copy(sfs[b], self_out.at[pl.ds(base + s_off, SCH)])

    pltpu.sync_copy(alpha_hbm, alpha_v)
    for i in range(NVR):
        sl = pl.ds(i * L, L)
        a0, a1, a2 = alpha_v[0, sl], alpha_v[1, sl], alpha_v[2, sl]
        m = jnp.maximum(jnp.maximum(a0, a1), a2)
        e0, e1, e2 = jnp.exp(a0 - m), jnp.exp(a1 - m), jnp.exp(a2 - m)
        inv = (1.0 / DEG) / (e0 + e1 + e2)
        w_v[0, sl] = e0 * inv
        w_v[1, sl] = e1 * inv
        w_v[2, sl] = e2 * inv

    zero = jnp.zeros((L,), jnp.float32)

    def zbody(i, c):
        for k in range(NVR):
            wagg_v[i, pl.ds(k * L, L)] = zero
        return c
    lax.fori_loop(0, BPW, zbody, 0)

    pltpu.sync_copy(neigh_hbm.at[wid], neigh_v)

    def fire(c, b):
        off = pl.multiple_of(c * (CH * DEG), CH * DEG)
        return pltpu.async_copy(
            feat_hbm.at[neigh_v.at[pl.ds(off, CH * DEG)]], sts[b], sems[b])

    def process(c, st):
        r = c // NCH
        wk = tuple(w_v[r, pl.ds(k * L, L)] for k in range(NVR))
        c0 = (c % NCH) * CH
        for j in range(CH):
            ci = c0 + j
            for k in range(NVR):
                sl = pl.ds(k * L, L)
                vals = [st[j * DEG + t, sl] for t in range(DEG)]
                while len(vals) > 1:
                    vals = [vals[2 * i] + vals[2 * i + 1]
                            for i in range(len(vals) // 2)]
                wagg_v[ci, sl] = wagg_v[ci, sl] + vals[0] * wk[k]

    for b in range(NBUF):
        fire(b, b)

    def main_body(p, carry):
        for b in range(NBUF):
            c = p * NBUF + b
            _wait_chunk(feat_hbm, sts[b], sems[b])
            process(c, sts[b])
            fire((c + NBUF) % TOT, b)
        return carry
    lax.fori_loop(0, TOT // NBUF, main_body, 0)

    for b in range(NBUF):
        _wait_chunk(feat_hbm, sts[b], sems[b])

    pltpu.sync_copy(wagg_v, wagg_out.at[pl.ds(base, BPW)])


def _wait_chunk(feat_hbm, st, sem):
    pltpu.make_async_copy(feat_hbm.at[pl.ds(0, CH * DEG)], st, sem).wait()


def _pad_idx(x, n_rows):
    x = x.astype(jnp.int32)
    pad = n_rows - x.shape[0]
    cfg = [(0, pad)] + [(0, 0)] * (x.ndim - 1)
    return jnp.pad(x, cfg)


def kernel(features, alpha, nodes, neigh1, neigh2, neigh3):
    features = features.astype(jnp.float32)
    alpha_t = alpha[D:, :].T.astype(jnp.float32)
    nodes_p = _pad_idx(nodes, NPAD).reshape(NW, BPW)
    neigh_p = jnp.stack([
        _pad_idx(n, NPAD).reshape(NW, BPW * DEG)
        for n in (neigh1, neigh2, neigh3)], axis=1)
    neigh_p = neigh_p.reshape(NW, NREL * BPW * DEG)

    mesh = plsc.VectorSubcoreMesh(core_axis_name="c", subcore_axis_name="s")
    f = pl.kernel(
        _sc_body,
        out_type=(jax.ShapeDtypeStruct((NPAD, D), jnp.float32),
                  jax.ShapeDtypeStruct((NPAD, D), jnp.float32)),
        mesh=mesh,
        scratch_types=(
            pltpu.VMEM((NREL, D), jnp.float32),
            pltpu.VMEM((NREL, D), jnp.float32),
            pltpu.VMEM((BPW,), jnp.int32),
            pltpu.VMEM((NREL * BPW * DEG,), jnp.int32),
            pltpu.VMEM((BPW, D), jnp.float32),
            pltpu.VMEM((CH * DEG, D), jnp.float32),
            pltpu.VMEM((CH * DEG, D), jnp.float32),
            pltpu.VMEM((SCH, D), jnp.float32),
            pltpu.VMEM((SCH, D), jnp.float32),
            pltpu.SemaphoreType.DMA,
            pltpu.SemaphoreType.DMA,
        ),
    )
    self_o, wagg_o = f(alpha_t, nodes_p, neigh_p, features)
    self_o = self_o[:BATCH]
    return jnp.concatenate([self_o, self_o, wagg_o[:BATCH]], axis=1)
